# baseline (speedup 1.0000x reference)
import functools
import jax
import jax.numpy as jnp
from jax import lax
from jax.experimental import pallas as pl
from jax.experimental.pallas import tpu as pltpu
from jax.experimental.pallas import tpu_sc as plsc

_THRESHOLD = 0.5
_NB_AVERAGE = 9
_OFFSET = _NB_AVERAGE // 2

_L = 16
_NSLICE = 23
_G = 8
_CHUNK = _G * _L
_NW = 32


def _sc_call(x_flat, fbins, rows_b, n_bins):
    rows_w = rows_b // _NW
    nchunks = rows_w // _CHUNK
    chunk_words = _CHUNK * n_bins
    mesh = plsc.VectorSubcoreMesh(core_axis_name="c", subcore_axis_name="s")

    @functools.partial(
        pl.kernel,
        mesh=mesh,
        out_type=[
            jax.ShapeDtypeStruct((rows_b,), jnp.float32),
            jax.ShapeDtypeStruct((rows_b,), jnp.float32),
        ],
        scratch_types=[
            pltpu.VMEM((2 * chunk_words + _L,), jnp.float32),
            pltpu.VMEM((384,), jnp.float32),
            pltpu.VMEM((_CHUNK,), jnp.float32),
            pltpu.VMEM((_CHUNK,), jnp.float32),
            pltpu.SemaphoreType.DMA,
            pltpu.SemaphoreType.DMA,
            pltpu.SemaphoreType.DMA,
        ],
        compiler_params=pltpu.CompilerParams(needs_layout_passes=False),
    )
    def k(x_hbm, fb_hbm, f_hbm, c_hbm, buf, fbv, fres, cres,
          sem_a, sem_b, sem_out):
        wid = lax.axis_index("s") * 2 + lax.axis_index("c")
        base_row = wid * rows_w
        pltpu.sync_copy(fb_hbm, fbv.at[pl.ds(0, 360)])
        lanes = lax.iota(jnp.int32, _L)
        zeros_f = jnp.zeros((_L,), jnp.float32)
        gmask = lanes < _NB_AVERAGE
        sems = (sem_a, sem_b)

        def start_fetch(ci, slot):
            row0 = base_row + ci * _CHUNK
            return pltpu.async_copy(
                x_hbm.at[pl.ds(row0 * n_bins, chunk_words)],
                buf.at[pl.ds(slot * chunk_words, chunk_words)],
                sems[slot],
            )

        def bfly(v, op):
            for s in (8, 4, 2, 1):
                perm = jnp.bitwise_xor(lanes, s)
                v = op(v, v.at[perm].get(mode="promise_in_bounds"))
            return v

        start_fetch(0, 0)

        def process_chunk(ci, slot):
            boff = slot * chunk_words
            row0c = base_row + ci * _CHUNK
            pltpu.make_async_copy(
                x_hbm.at[pl.ds(row0c * n_bins, chunk_words)],
                buf.at[pl.ds(boff, chunk_words)],
                sems[slot],
            ).wait()

            @pl.when(ci + 1 < nchunks)
            def _():
                start_fetch(ci + 1, 1 - slot)

            def group_body(g, carry2):
                goff = boff + g * (_L * n_bins)
                psacc = zeros_f
                wsacc = zeros_f
                macc = zeros_f
                for rr0 in range(0, _L, 4):
                    roffs = [goff + (rr0 + j) * n_bins for j in range(4)]
                    ms = [jnp.full((_L,), -jnp.inf, dtype=jnp.float32)
                          for _ in range(4)]
                    bidxs = [jnp.zeros((_L,), jnp.int32) for _ in range(4)]
                    for kk in range(_NSLICE):
                        for j in range(4):
                            v = buf[pl.ds(roffs[j] + kk * _L, _L)]
                            if kk == _NSLICE - 1:
                                v = jnp.where(lanes < (n_bins - kk * _L), v,
                                              -jnp.inf)
                            upd = v > ms[j]
                            ms[j] = jnp.where(upd, v, ms[j])
                            bidxs[j] = jnp.where(upd, kk * _L + lanes,
                                                 bidxs[j])
                    for j in range(4):
                        rr = rr0 + j
                        mmax = bfly(ms[j], jnp.maximum)
                        cand = jnp.where(ms[j] == mmax, bidxs[j], n_bins)
                        center = bfly(cand, jnp.minimum)
                        start = jnp.clip(center - _OFFSET,
                                         0, n_bins - _NB_AVERAGE)
                        gidx = start + lanes
                        w = plsc.load_gather(buf, [roffs[j] + gidx])
                        cc = plsc.load_gather(fbv, [gidx])
                        w = jnp.where(gmask, w, 0.0)
                        cc = jnp.where(gmask, cc, 0.0)
                        wsum = bfly(w, jnp.add)
                        psum = bfly(w * cc, jnp.add)
                        lane_rr = lanes == rr
                        psacc = jnp.where(lane_rr, psum, psacc)
                        wsacc = jnp.where(lane_rr, wsum, wsacc)
                        macc = jnp.where(lane_rr, mmax, macc)
                fv = psacc / wsacc
                voiced = macc > _THRESHOLD
                fres[pl.ds(g * _L, _L)] = jnp.where(voiced, fv, 0.0)
                cres[pl.ds(g * _L, _L)] = jnp.where(voiced, macc, 1.0 - macc)
                return carry2

            lax.fori_loop(0, _G, group_body, 0, unroll=False)
            cp_f = pltpu.async_copy(fres, f_hbm.at[pl.ds(row0c, _CHUNK)],
                                    sem_out)
            cp_c = pltpu.async_copy(cres, c_hbm.at[pl.ds(row0c, _CHUNK)],
                                    sem_out)
            cp_f.wait()
            cp_c.wait()

        def chunk_pair_body(cp, carry):
            process_chunk(2 * cp, 0)
            process_chunk(2 * cp + 1, 1)
            return carry

        lax.fori_loop(0, nchunks // 2, chunk_pair_body, 0, unroll=False)

    return k(x_flat, fbins)


def kernel(inputs, fbins):
    b, t, n_bins = inputs.shape
    rows = b * t
    x_flat = inputs.reshape(rows * n_bins)
    f, c = _sc_call(x_flat, fbins, rows, n_bins)
    return jnp.stack([f.reshape(b, t), c.reshape(b, t)], axis=2)
